# baseline (speedup 1.0000x reference)
import dataclasses
import functools

import jax
import jax.numpy as jnp
from jax import lax
from jax.experimental import pallas as pl
from jax.experimental.pallas import tpu as pltpu
from jax.experimental.pallas import tpu_sc as plsc

N = 10000
D = 256
E = 160000
NC, NS, L = 2, 16, 16
CH = 128
E_PAD = 163840
ROWS = E_PAD // CH
ROWS_T32 = ROWS // (NC * NS)
ROWS_T16 = ROWS // NS
HALF = N // 2
ACC_ROWS = 5120
SLAB = ACC_ROWS // NS
DH = D // 2
DEG_LEN = 10240
DEG_SLAB = DEG_LEN // NS

_mesh = plsc.VectorSubcoreMesh(core_axis_name="c", subcore_axis_name="s")

_cp_no_layout = pltpu.CompilerParams()
if "needs_layout_passes" in pltpu.CompilerParams.__dataclass_fields__:
    _cp_no_layout = dataclasses.replace(_cp_no_layout, needs_layout_passes=False)


CAP = ROWS_T16 * CH
CAP_PAD = CAP + CH
CH2 = 80
NCH2 = CAP // CH2


def _sc_compact(src_rows, dst_rows, val_rows):
    @functools.partial(
        pl.kernel,
        out_type=[jax.ShapeDtypeStruct((NC, NS, CAP), jnp.int32),
                  jax.ShapeDtypeStruct((NC, NS, CAP), jnp.int32),
                  jax.ShapeDtypeStruct((NC, DEG_LEN), jnp.float32)],
        mesh=_mesh,
        scratch_types=[
            pltpu.VMEM((ROWS_T16, CH), jnp.int32),
            pltpu.VMEM((ROWS_T16, CH), jnp.int32),
            pltpu.VMEM((CAP_PAD,), jnp.int32),
            pltpu.VMEM((CAP_PAD,), jnp.int32),
            pltpu.VMEM((ROWS_T32, CH), jnp.float32),
            pltpu.VMEM((DEG_SLAB,), jnp.float32),
            pltpu.VMEM_SHARED((DEG_LEN,), jnp.float32),
        ],
        compiler_params=_cp_no_layout,
    )
    def k(src_hbm, dst_hbm, val_hbm, srcc_hbm, dstc_hbm, deg_hbm,
          src_v, dst_v, src_c, dst_c, val_v, zbuf, deg_sh):
        c = lax.axis_index("c")
        s = lax.axis_index("s")
        wid = c * NS + s
        lo = c * HALF

        @pl.loop(0, DEG_SLAB, step=L)
        def _(i):
            zbuf[pl.ds(i, L)] = jnp.zeros((L,), jnp.float32)

        pltpu.sync_copy(zbuf, deg_sh.at[pl.ds(s * DEG_SLAB, DEG_SLAB)])
        pltpu.sync_copy(src_hbm.at[pl.ds(s * ROWS_T16, ROWS_T16)], src_v)
        pltpu.sync_copy(dst_hbm.at[pl.ds(s * ROWS_T16, ROWS_T16)], dst_v)
        pltpu.sync_copy(
            val_hbm.at[pl.ds(s * ROWS_T16 + c * ROWS_T32, ROWS_T32)], val_v)
        plsc.subcore_barrier()

        d0 = c * ROWS_T32
        @pl.loop(0, ROWS_T32)
        def _(j):
            pltpu.sync_copy(val_v.at[j], deg_sh.at[src_v.at[d0 + j]],
                            add=True)

        @pl.loop(0, CAP_PAD, step=L)
        def _(i):
            src_c[pl.ds(i, L)] = jnp.zeros((L,), jnp.int32)
            dst_c[pl.ds(i, L)] = jnp.full((L,), HALF, jnp.int32)

        def _compact_group(i, off, j):
            sv = src_v[j, pl.ds(i, L)]
            dv = dst_v[j, pl.ds(i, L)] - lo
            ok = (dv >= 0) & (dv < HALF)
            plsc.store_compressed(src_c.at[pl.ds(off, L)], sv, mask=ok)
            plsc.store_compressed(dst_c.at[pl.ds(off, L)], dv, mask=ok)
            return off + jnp.sum(ok.astype(jnp.int32))

        def _compact_row(j, off):
            return pl.loop(0, CH, step=L, init_carry=off)(
                lambda i, o: _compact_group(i, o, j))

        pl.loop(0, ROWS_T16, init_carry=jnp.int32(0))(_compact_row)
        pltpu.sync_copy(src_c.at[pl.ds(0, CAP)], srcc_hbm.at[c, s])
        pltpu.sync_copy(dst_c.at[pl.ds(0, CAP)], dstc_hbm.at[c, s])
        plsc.subcore_barrier()
        pltpu.sync_copy(deg_sh.at[pl.ds(s * DEG_SLAB, DEG_SLAB)],
                        deg_hbm.at[c, pl.ds(s * DEG_SLAB, DEG_SLAB)])

    return k(src_rows, dst_rows, val_rows)


def _sc_scatter(y0, y1, srcc, dstc):
    @functools.partial(
        pl.kernel,
        out_type=[jax.ShapeDtypeStruct((NC, ACC_ROWS, DH), jnp.float32),
                  jax.ShapeDtypeStruct((NC, ACC_ROWS, DH), jnp.float32)],
        mesh=_mesh,
        scratch_types=[
            pltpu.VMEM((NCH2, CH2), jnp.int32),
            pltpu.VMEM((NCH2, CH2), jnp.int32),
            pltpu.VMEM((CH2, DH), jnp.float32),
            pltpu.VMEM((CH2, DH), jnp.float32),
            [pltpu.SemaphoreType.DMA] * 2,
            [pltpu.SemaphoreType.DMA] * 2,
            pltpu.VMEM_SHARED((ACC_ROWS, DH), jnp.float32),
        ],
        compiler_params=_cp_no_layout,
    )
    def k(y0_hbm, y1_hbm, srcc_hbm, dstc_hbm, o0_hbm, o1_hbm,
          src_c, dst_c,
          buf0, buf1, gsems, ssems, acc_sh):
        c = lax.axis_index("c")
        s = lax.axis_index("s")
        base = s * SLAB

        def zero_buf0():
            @pl.loop(0, CH2)
            def _(i):
                @pl.loop(0, DH, step=L)
                def _(k2):
                    buf0[i, pl.ds(k2, L)] = jnp.zeros((L,), jnp.float32)

        def zero_slab():
            for r in range(0, SLAB, CH2):
                pltpu.sync_copy(buf0, acc_sh.at[pl.ds(base + r, CH2)])

        zero_buf0()
        zero_slab()
        pltpu.sync_copy(srcc_hbm.at[c, s], src_c)
        pltpu.sync_copy(dstc_hbm.at[c, s], dst_c)

        def _count_row(j, m):
            t = jnp.zeros((L,), jnp.int32)
            for i in range(0, CH2, L):
                t = t + (dst_c[j, pl.ds(i, L)] < HALF).astype(jnp.int32)
            return m + jnp.sum(t)

        off = pl.loop(0, NCH2, init_carry=jnp.int32(0))(_count_row)

        nch = (off + CH2 - 1) // CH2
        bound = jnp.maximum((nch + 1) // 2 * 2, 2)
        bufs = (buf0, buf1)

        for p, (y_hbm, o_hbm) in enumerate(((y0_hbm, o0_hbm),
                                            (y1_hbm, o1_hbm))):
            plsc.subcore_barrier()

            def start_gather(j, k2):
                pltpu.async_copy(y_hbm.at[src_c.at[j]], bufs[k2], gsems[k2])

            def wait_gather(k2):
                pltpu.make_async_copy(y_hbm.at[pl.ds(0, CH2)], bufs[k2],
                                      gsems[k2]).wait()

            def start_scatter(j, k2):
                pltpu.async_copy(bufs[k2], acc_sh.at[dst_c.at[j]], ssems[k2],
                                 add=True)

            def wait_scatter(k2):
                pltpu.make_async_copy(bufs[k2], acc_sh.at[pl.ds(0, CH2)],
                                      ssems[k2]).wait()

            start_gather(0, 0)

            @pl.loop(0, bound, step=2)
            def _(j):
                start_gather(j + 1, 1)
                wait_gather(0)
                start_scatter(j, 0)
                wait_scatter(0)

                @pl.when(j + 2 < bound)
                def _():
                    start_gather(j + 2, 0)

                wait_gather(1)
                start_scatter(j + 1, 1)
                wait_scatter(1)

            plsc.subcore_barrier()
            pltpu.sync_copy(acc_sh.at[pl.ds(base, SLAB)],
                            o_hbm.at[c, pl.ds(base, SLAB)])
            if p == 0:
                zero_buf0()
                zero_slab()

    return k(y0, y1, srcc, dstc)


CH_S = 64
SROWS = E_PAD // CH_S
NCH_S = SROWS // (NC * NS)


def _sc_score(u, hr, s_rows, t_rows):
    @functools.partial(
        pl.kernel,
        out_type=jax.ShapeDtypeStruct((SROWS, CH_S), jnp.float32),
        mesh=_mesh,
        scratch_types=[
            pltpu.VMEM((NCH_S, CH_S), jnp.int32),
            pltpu.VMEM((NCH_S, CH_S), jnp.int32),
            pltpu.VMEM((CH_S, D), jnp.float32),
            pltpu.VMEM((CH_S, D), jnp.float32),
            pltpu.VMEM((CH_S, D), jnp.float32),
            pltpu.VMEM((CH_S, D), jnp.float32),
            pltpu.VMEM((NCH_S, CH_S), jnp.float32),
            pltpu.VMEM((L, L), jnp.float32),
            pltpu.SemaphoreType.DMA,
            pltpu.SemaphoreType.DMA,
            pltpu.SemaphoreType.DMA,
            pltpu.SemaphoreType.DMA,
        ],
        compiler_params=_cp_no_layout,
    )
    def k(u_hbm, hr_hbm, s_hbm, t_hbm, out_hbm,
          s_v, t_v, bufA0, bufB0, bufA1, bufB1, pbuf, pstage,
          semA0, semB0, semA1, semB1):
        c = lax.axis_index("c")
        s = lax.axis_index("s")
        wid = c * NS + s
        pltpu.sync_copy(s_hbm.at[pl.ds(wid * NCH_S, NCH_S)], s_v)
        pltpu.sync_copy(t_hbm.at[pl.ds(wid * NCH_S, NCH_S)], t_v)
        rows16 = lax.iota(jnp.int32, L)

        def start(j, bA, bB, sA, sB):
            pltpu.async_copy(u_hbm.at[s_v.at[j]], bA, sA)
            pltpu.async_copy(hr_hbm.at[t_v.at[j]], bB, sB)

        def wait(bA, bB, sA, sB):
            pltpu.make_async_copy(u_hbm.at[pl.ds(0, CH_S)], bA, sA).wait()
            pltpu.make_async_copy(hr_hbm.at[pl.ds(0, CH_S)], bB, sB).wait()

        def compute(j, bA, bB):
            @pl.loop(0, CH_S, step=L)
            def _(g):
                @pl.loop(0, L)
                def _(e):
                    ea = g + e
                    acc0 = jnp.zeros((L,), jnp.float32)
                    acc1 = jnp.zeros((L,), jnp.float32)
                    for kk in range(0, L, 2):
                        acc0 = acc0 + (bA[ea, pl.ds(kk * L, L)] *
                                       bB[ea, pl.ds(kk * L, L)])
                        acc1 = acc1 + (bA[ea, pl.ds((kk + 1) * L, L)] *
                                       bB[ea, pl.ds((kk + 1) * L, L)])
                    pstage[e, pl.ds(0, L)] = acc0 + acc1

                accT = jnp.zeros((L,), jnp.float32)
                for kk in range(L):
                    cols = jnp.full((L,), kk, jnp.int32)
                    accT = accT + plsc.load_gather(pstage, [rows16, cols])
                pbuf[j, pl.ds(g, L)] = accT

        start(0, bufA0, bufB0, semA0, semB0)

        @pl.loop(0, NCH_S, step=2)
        def _(j):
            start(j + 1, bufA1, bufB1, semA1, semB1)
            wait(bufA0, bufB0, semA0, semB0)
            compute(j, bufA0, bufB0)

            @pl.when(j + 2 < NCH_S)
            def _():
                start(j + 2, bufA0, bufB0, semA0, semB0)

            wait(bufA1, bufB1, semA1, semB1)
            compute(j + 1, bufA1, bufB1)

        pltpu.sync_copy(pbuf, out_hbm.at[pl.ds(wid * NCH_S, NCH_S)])

    return k(u, hr, s_rows, t_rows)


def _tc_prescale(degsum, x):
    BR = 1000

    def body(deg_ref, x_ref, dis_ref, y0_ref, y1_ref):
        dv = deg_ref[...]
        dis = jnp.where(dv > 0, lax.rsqrt(dv), 0.0)
        dis_ref[...] = dis
        y = dis * x_ref[...]
        y0_ref[...] = y[:, :DH]
        y1_ref[...] = y[:, DH:]

    return pl.pallas_call(
        body,
        grid=(N // BR,),
        in_specs=[pl.BlockSpec((BR, 1), lambda i: (i, 0)),
                  pl.BlockSpec((BR, D), lambda i: (i, 0))],
        out_specs=[pl.BlockSpec((BR, 1), lambda i: (i, 0)),
                   pl.BlockSpec((BR, DH), lambda i: (i, 0)),
                   pl.BlockSpec((BR, DH), lambda i: (i, 0))],
        out_shape=[jax.ShapeDtypeStruct((N, 1), jnp.float32),
                   jax.ShapeDtypeStruct((N, DH), jnp.float32),
                   jax.ShapeDtypeStruct((N, DH), jnp.float32)],
    )(degsum, x)


def _tc_main(x, acc0, acc1, dis, Wx, Wt, bc, wpost):
    BR = 1000

    def body(x_ref, a0_ref, a1_ref, dis_ref, wx_ref, wt_ref, b_ref, wp_ref,
             h_ref, u_ref, hr_ref):
        t1 = jnp.concatenate([a0_ref[0], a1_ref[0]], axis=1) * (-dis_ref[...])
        A = (jnp.dot(x_ref[...].astype(jnp.bfloat16),
                     wx_ref[...].astype(jnp.bfloat16),
                     preferred_element_type=jnp.float32) +
             jnp.dot(t1.astype(jnp.bfloat16),
                     wt_ref[...].astype(jnp.bfloat16),
                     preferred_element_type=jnp.float32) +
             b_ref[...])
        z = jax.nn.sigmoid(A[:, :D])
        ht = jnp.tanh(A[:, D:])
        h = (1.0 - z) * ht
        h_ref[...] = h
        hrv = jnp.maximum(h, 0.0)
        hr_ref[...] = hrv
        u_ref[...] = hrv * wp_ref[...]

    return pl.pallas_call(
        body,
        grid=(N // BR,),
        in_specs=[pl.BlockSpec((BR, D), lambda i: (i, 0)),
                  pl.BlockSpec((1, BR, DH), lambda i: (i // 5, i % 5, 0)),
                  pl.BlockSpec((1, BR, DH), lambda i: (i // 5, i % 5, 0)),
                  pl.BlockSpec((BR, 1), lambda i: (i, 0)),
                  pl.BlockSpec((D, 2 * D), lambda i: (0, 0)),
                  pl.BlockSpec((D, 2 * D), lambda i: (0, 0)),
                  pl.BlockSpec((1, 2 * D), lambda i: (0, 0)),
                  pl.BlockSpec((1, D), lambda i: (0, 0))],
        out_specs=[pl.BlockSpec((BR, D), lambda i: (i, 0)),
                   pl.BlockSpec((BR, D), lambda i: (i, 0)),
                   pl.BlockSpec((BR, D), lambda i: (i, 0))],
        out_shape=[jax.ShapeDtypeStruct((N, D), jnp.float32),
                   jax.ShapeDtypeStruct((N, D), jnp.float32),
                   jax.ShapeDtypeStruct((N, D), jnp.float32)],
    )(x, acc0, acc1, dis, Wx, Wt, bc, wpost)


def kernel(x, edge_index, edge_label_index, W_xz, b_xz, W_hz, b_hz,
           W_xr, b_xr, W_hr, b_hr, W_xh, b_xh, W_hh, b_hh, W_post, b_post):
    pad = E_PAD - E
    i32 = jnp.int32
    src = edge_index[0].astype(i32)
    dst = edge_index[1].astype(i32)
    src_rows = jnp.concatenate([src, jnp.zeros((pad,), i32)]).reshape(ROWS, CH)
    dst_rows = jnp.concatenate([dst, jnp.full((pad,), -1, i32)]).reshape(ROWS, CH)
    val_rows = jnp.concatenate([jnp.ones((E,), jnp.float32),
                                jnp.zeros((pad,), jnp.float32)]).reshape(ROWS, CH)
    s_rows = jnp.concatenate([edge_label_index[0].astype(i32),
                              jnp.zeros((pad,), i32)]).reshape(SROWS, CH_S)
    t_rows = jnp.concatenate([edge_label_index[1].astype(i32),
                              jnp.zeros((pad,), i32)]).reshape(SROWS, CH_S)

    srcc, dstc, degp = _sc_compact(src_rows, dst_rows, val_rows)
    degsum = (degp[0] + degp[1])[:N].reshape(N, 1)
    dis, y0, y1 = _tc_prescale(degsum, x)

    srcc = srcc.reshape(NC, NS, NCH2, CH2)
    dstc = dstc.reshape(NC, NS, NCH2, CH2)
    acc0, acc1 = _sc_scatter(y0, y1, srcc, dstc)

    Wx = jnp.concatenate([W_xz[0], W_xh[0]], axis=1)
    Wt = jnp.concatenate([W_xz[1], W_xh[1]], axis=1)
    bc = jnp.concatenate([b_xz + b_hz, b_xh + b_hh]).reshape(1, 2 * D)
    wpost = W_post.sum(axis=1).reshape(1, D)
    h, u, hr = _tc_main(x, acc0, acc1, dis, Wx, Wt, bc, wpost)

    sc = _sc_score(u, hr, s_rows, t_rows)
    pred = sc.reshape(-1)[:E] + b_post.sum()
    return (pred, h)


# speedup vs baseline: 1.0504x; 1.0504x over previous
import dataclasses
import functools

import jax
import jax.numpy as jnp
from jax import lax
from jax.experimental import pallas as pl
from jax.experimental.pallas import tpu as pltpu
from jax.experimental.pallas import tpu_sc as plsc

N = 10000
D = 256
E = 160000
NC, NS, L = 2, 16, 16
CH = 128
E_PAD = 163840
ROWS = E_PAD // CH
ROWS_T32 = ROWS // (NC * NS)
ROWS_T16 = ROWS // NS
HALF = N // 2
ACC_ROWS = 5120
SLAB = ACC_ROWS // NS
DH = D // 2
DEG_LEN = 10240
DEG_SLAB = DEG_LEN // NS

_mesh = plsc.VectorSubcoreMesh(core_axis_name="c", subcore_axis_name="s")

_cp_no_layout = pltpu.CompilerParams()
if "needs_layout_passes" in pltpu.CompilerParams.__dataclass_fields__:
    _cp_no_layout = dataclasses.replace(_cp_no_layout, needs_layout_passes=False)


CAP = ROWS_T16 * CH
CAP_PAD = CAP + CH
CH2 = 64
NCH2 = CAP // CH2


def _sc_compact(src_rows, dst_rows, val_rows):
    @functools.partial(
        pl.kernel,
        out_type=[jax.ShapeDtypeStruct((NC, NS, CAP), jnp.int32),
                  jax.ShapeDtypeStruct((NC, NS, CAP), jnp.int32),
                  jax.ShapeDtypeStruct((NC, DEG_LEN), jnp.float32)],
        mesh=_mesh,
        scratch_types=[
            pltpu.VMEM((ROWS_T16, CH), jnp.int32),
            pltpu.VMEM((ROWS_T16, CH), jnp.int32),
            pltpu.VMEM((CAP_PAD,), jnp.int32),
            pltpu.VMEM((CAP_PAD,), jnp.int32),
            pltpu.VMEM((ROWS_T32, CH), jnp.float32),
            pltpu.VMEM((DEG_SLAB,), jnp.float32),
            pltpu.VMEM_SHARED((DEG_LEN,), jnp.float32),
        ],
        compiler_params=_cp_no_layout,
    )
    def k(src_hbm, dst_hbm, val_hbm, srcc_hbm, dstc_hbm, deg_hbm,
          src_v, dst_v, src_c, dst_c, val_v, zbuf, deg_sh):
        c = lax.axis_index("c")
        s = lax.axis_index("s")
        wid = c * NS + s
        lo = c * HALF

        @pl.loop(0, DEG_SLAB, step=L)
        def _(i):
            zbuf[pl.ds(i, L)] = jnp.zeros((L,), jnp.float32)

        pltpu.sync_copy(zbuf, deg_sh.at[pl.ds(s * DEG_SLAB, DEG_SLAB)])
        pltpu.sync_copy(src_hbm.at[pl.ds(s * ROWS_T16, ROWS_T16)], src_v)
        pltpu.sync_copy(dst_hbm.at[pl.ds(s * ROWS_T16, ROWS_T16)], dst_v)
        pltpu.sync_copy(
            val_hbm.at[pl.ds(s * ROWS_T16 + c * ROWS_T32, ROWS_T32)], val_v)
        plsc.subcore_barrier()

        d0 = c * ROWS_T32
        @pl.loop(0, ROWS_T32)
        def _(j):
            pltpu.sync_copy(val_v.at[j], deg_sh.at[src_v.at[d0 + j]],
                            add=True)

        @pl.loop(0, CAP_PAD, step=L)
        def _(i):
            src_c[pl.ds(i, L)] = jnp.zeros((L,), jnp.int32)
            dst_c[pl.ds(i, L)] = jnp.full((L,), HALF, jnp.int32)

        def _compact_group(i, off, j):
            sv = src_v[j, pl.ds(i, L)]
            dv = dst_v[j, pl.ds(i, L)] - lo
            ok = (dv >= 0) & (dv < HALF)
            plsc.store_compressed(src_c.at[pl.ds(off, L)], sv, mask=ok)
            plsc.store_compressed(dst_c.at[pl.ds(off, L)], dv, mask=ok)
            return off + jnp.sum(ok.astype(jnp.int32))

        def _compact_row(j, off):
            return pl.loop(0, CH, step=L, init_carry=off)(
                lambda i, o: _compact_group(i, o, j))

        pl.loop(0, ROWS_T16, init_carry=jnp.int32(0))(_compact_row)
        pltpu.sync_copy(src_c.at[pl.ds(0, CAP)], srcc_hbm.at[c, s])
        pltpu.sync_copy(dst_c.at[pl.ds(0, CAP)], dstc_hbm.at[c, s])
        plsc.subcore_barrier()
        pltpu.sync_copy(deg_sh.at[pl.ds(s * DEG_SLAB, DEG_SLAB)],
                        deg_hbm.at[c, pl.ds(s * DEG_SLAB, DEG_SLAB)])

    return k(src_rows, dst_rows, val_rows)


def _sc_scatter(y0, y1, srcc, dstc):
    @functools.partial(
        pl.kernel,
        out_type=[jax.ShapeDtypeStruct((NC, ACC_ROWS, DH), jnp.float32),
                  jax.ShapeDtypeStruct((NC, ACC_ROWS, DH), jnp.float32)],
        mesh=_mesh,
        scratch_types=[
            pltpu.VMEM((NCH2, CH2), jnp.int32),
            pltpu.VMEM((NCH2, CH2), jnp.int32),
            pltpu.VMEM((CH2, DH), jnp.float32),
            pltpu.VMEM((CH2, DH), jnp.float32),
            [pltpu.SemaphoreType.DMA] * 2,
            [pltpu.SemaphoreType.DMA] * 2,
            pltpu.VMEM_SHARED((ACC_ROWS, DH), jnp.float32),
        ],
        compiler_params=_cp_no_layout,
    )
    def k(y0_hbm, y1_hbm, srcc_hbm, dstc_hbm, o0_hbm, o1_hbm,
          src_c, dst_c,
          buf0, buf1, gsems, ssems, acc_sh):
        c = lax.axis_index("c")
        s = lax.axis_index("s")
        base = s * SLAB

        def zero_buf0():
            @pl.loop(0, CH2)
            def _(i):
                @pl.loop(0, DH, step=L)
                def _(k2):
                    buf0[i, pl.ds(k2, L)] = jnp.zeros((L,), jnp.float32)

        def zero_slab():
            for r in range(0, SLAB, CH2):
                pltpu.sync_copy(buf0, acc_sh.at[pl.ds(base + r, CH2)])

        zero_buf0()
        zero_slab()
        pltpu.sync_copy(srcc_hbm.at[c, s], src_c)
        pltpu.sync_copy(dstc_hbm.at[c, s], dst_c)

        def _count_row(j, m):
            t = jnp.zeros((L,), jnp.int32)
            for i in range(0, CH2, L):
                t = t + (dst_c[j, pl.ds(i, L)] < HALF).astype(jnp.int32)
            return m + jnp.sum(t)

        off = pl.loop(0, NCH2, init_carry=jnp.int32(0))(_count_row)

        nch = (off + CH2 - 1) // CH2
        bound = jnp.maximum((nch + 1) // 2 * 2, 2)
        bufs = (buf0, buf1)

        for p, (y_hbm, o_hbm) in enumerate(((y0_hbm, o0_hbm),
                                            (y1_hbm, o1_hbm))):
            plsc.subcore_barrier()

            def start_gather(j, k2):
                pltpu.async_copy(y_hbm.at[src_c.at[j]], bufs[k2], gsems[k2])

            def wait_gather(k2):
                pltpu.make_async_copy(y_hbm.at[pl.ds(0, CH2)], bufs[k2],
                                      gsems[k2]).wait()

            def start_scatter(j, k2):
                pltpu.async_copy(bufs[k2], acc_sh.at[dst_c.at[j]], ssems[k2],
                                 add=True)

            def wait_scatter(k2):
                pltpu.make_async_copy(bufs[k2], acc_sh.at[pl.ds(0, CH2)],
                                      ssems[k2]).wait()

            start_gather(0, 0)

            @pl.loop(0, bound, step=2)
            def _(j):
                start_gather(j + 1, 1)
                wait_gather(0)
                start_scatter(j, 0)
                wait_scatter(0)

                @pl.when(j + 2 < bound)
                def _():
                    start_gather(j + 2, 0)

                wait_gather(1)
                start_scatter(j + 1, 1)
                wait_scatter(1)

            plsc.subcore_barrier()
            pltpu.sync_copy(acc_sh.at[pl.ds(base, SLAB)],
                            o_hbm.at[c, pl.ds(base, SLAB)])
            if p == 0:
                zero_buf0()
                zero_slab()

    return k(y0, y1, srcc, dstc)


CH_S = 64
SROWS = E_PAD // CH_S
NCH_S = SROWS // (NC * NS)


def _sc_score(u, hr, s_rows, t_rows):
    @functools.partial(
        pl.kernel,
        out_type=jax.ShapeDtypeStruct((SROWS, CH_S), jnp.float32),
        mesh=_mesh,
        scratch_types=[
            pltpu.VMEM((NCH_S, CH_S), jnp.int32),
            pltpu.VMEM((NCH_S, CH_S), jnp.int32),
            pltpu.VMEM((CH_S, D), jnp.float32),
            pltpu.VMEM((CH_S, D), jnp.float32),
            pltpu.VMEM((CH_S, D), jnp.float32),
            pltpu.VMEM((CH_S, D), jnp.float32),
            pltpu.VMEM((NCH_S, CH_S), jnp.float32),
            pltpu.VMEM((L, L), jnp.float32),
            pltpu.SemaphoreType.DMA,
            pltpu.SemaphoreType.DMA,
            pltpu.SemaphoreType.DMA,
            pltpu.SemaphoreType.DMA,
        ],
        compiler_params=_cp_no_layout,
    )
    def k(u_hbm, hr_hbm, s_hbm, t_hbm, out_hbm,
          s_v, t_v, bufA0, bufB0, bufA1, bufB1, pbuf, pstage,
          semA0, semB0, semA1, semB1):
        c = lax.axis_index("c")
        s = lax.axis_index("s")
        wid = c * NS + s
        pltpu.sync_copy(s_hbm.at[pl.ds(wid * NCH_S, NCH_S)], s_v)
        pltpu.sync_copy(t_hbm.at[pl.ds(wid * NCH_S, NCH_S)], t_v)
        rows16 = lax.iota(jnp.int32, L)

        def start(j, bA, bB, sA, sB):
            pltpu.async_copy(u_hbm.at[s_v.at[j]], bA, sA)
            pltpu.async_copy(hr_hbm.at[t_v.at[j]], bB, sB)

        def wait(bA, bB, sA, sB):
            pltpu.make_async_copy(u_hbm.at[pl.ds(0, CH_S)], bA, sA).wait()
            pltpu.make_async_copy(hr_hbm.at[pl.ds(0, CH_S)], bB, sB).wait()

        def compute(j, bA, bB):
            @pl.loop(0, CH_S, step=L)
            def _(g):
                @pl.loop(0, L)
                def _(e):
                    ea = g + e
                    acc0 = jnp.zeros((L,), jnp.float32)
                    acc1 = jnp.zeros((L,), jnp.float32)
                    for kk in range(0, L, 2):
                        acc0 = acc0 + (bA[ea, pl.ds(kk * L, L)] *
                                       bB[ea, pl.ds(kk * L, L)])
                        acc1 = acc1 + (bA[ea, pl.ds((kk + 1) * L, L)] *
                                       bB[ea, pl.ds((kk + 1) * L, L)])
                    pstage[e, pl.ds(0, L)] = acc0 + acc1

                accT = jnp.zeros((L,), jnp.float32)
                for kk in range(L):
                    cols = jnp.full((L,), kk, jnp.int32)
                    accT = accT + plsc.load_gather(pstage, [rows16, cols])
                pbuf[j, pl.ds(g, L)] = accT

        start(0, bufA0, bufB0, semA0, semB0)

        @pl.loop(0, NCH_S, step=2)
        def _(j):
            start(j + 1, bufA1, bufB1, semA1, semB1)
            wait(bufA0, bufB0, semA0, semB0)
            compute(j, bufA0, bufB0)

            @pl.when(j + 2 < NCH_S)
            def _():
                start(j + 2, bufA0, bufB0, semA0, semB0)

            wait(bufA1, bufB1, semA1, semB1)
            compute(j + 1, bufA1, bufB1)

        pltpu.sync_copy(pbuf, out_hbm.at[pl.ds(wid * NCH_S, NCH_S)])

    return k(u, hr, s_rows, t_rows)


def _tc_prescale(degsum, x):
    BR = 1000

    def body(deg_ref, x_ref, dis_ref, y0_ref, y1_ref):
        dv = deg_ref[...]
        dis = jnp.where(dv > 0, lax.rsqrt(dv), 0.0)
        dis_ref[...] = dis
        y = dis * x_ref[...]
        y0_ref[...] = y[:, :DH]
        y1_ref[...] = y[:, DH:]

    return pl.pallas_call(
        body,
        grid=(N // BR,),
        in_specs=[pl.BlockSpec((BR, 1), lambda i: (i, 0)),
                  pl.BlockSpec((BR, D), lambda i: (i, 0))],
        out_specs=[pl.BlockSpec((BR, 1), lambda i: (i, 0)),
                   pl.BlockSpec((BR, DH), lambda i: (i, 0)),
                   pl.BlockSpec((BR, DH), lambda i: (i, 0))],
        out_shape=[jax.ShapeDtypeStruct((N, 1), jnp.float32),
                   jax.ShapeDtypeStruct((N, DH), jnp.float32),
                   jax.ShapeDtypeStruct((N, DH), jnp.float32)],
    )(degsum, x)


def _tc_main(x, acc0, acc1, dis, Wx, Wt, bc, wpost):
    BR = 1000

    def body(x_ref, a0_ref, a1_ref, dis_ref, wx_ref, wt_ref, b_ref, wp_ref,
             h_ref, u_ref, hr_ref):
        t1 = jnp.concatenate([a0_ref[0], a1_ref[0]], axis=1) * (-dis_ref[...])
        A = (jnp.dot(x_ref[...], wx_ref[...],
                     preferred_element_type=jnp.float32) +
             jnp.dot(t1, wt_ref[...], preferred_element_type=jnp.float32) +
             b_ref[...])
        z = jax.nn.sigmoid(A[:, :D])
        ht = jnp.tanh(A[:, D:])
        h = (1.0 - z) * ht
        h_ref[...] = h
        hrv = jnp.maximum(h, 0.0)
        hr_ref[...] = hrv
        u_ref[...] = hrv * wp_ref[...]

    return pl.pallas_call(
        body,
        grid=(N // BR,),
        in_specs=[pl.BlockSpec((BR, D), lambda i: (i, 0)),
                  pl.BlockSpec((1, BR, DH), lambda i: (i // 5, i % 5, 0)),
                  pl.BlockSpec((1, BR, DH), lambda i: (i // 5, i % 5, 0)),
                  pl.BlockSpec((BR, 1), lambda i: (i, 0)),
                  pl.BlockSpec((D, 2 * D), lambda i: (0, 0)),
                  pl.BlockSpec((D, 2 * D), lambda i: (0, 0)),
                  pl.BlockSpec((1, 2 * D), lambda i: (0, 0)),
                  pl.BlockSpec((1, D), lambda i: (0, 0))],
        out_specs=[pl.BlockSpec((BR, D), lambda i: (i, 0)),
                   pl.BlockSpec((BR, D), lambda i: (i, 0)),
                   pl.BlockSpec((BR, D), lambda i: (i, 0))],
        out_shape=[jax.ShapeDtypeStruct((N, D), jnp.float32),
                   jax.ShapeDtypeStruct((N, D), jnp.float32),
                   jax.ShapeDtypeStruct((N, D), jnp.float32)],
    )(x, acc0, acc1, dis, Wx, Wt, bc, wpost)


def kernel(x, edge_index, edge_label_index, W_xz, b_xz, W_hz, b_hz,
           W_xr, b_xr, W_hr, b_hr, W_xh, b_xh, W_hh, b_hh, W_post, b_post):
    pad = E_PAD - E
    i32 = jnp.int32
    src = edge_index[0].astype(i32)
    dst = edge_index[1].astype(i32)
    src_rows = jnp.concatenate([src, jnp.zeros((pad,), i32)]).reshape(ROWS, CH)
    dst_rows = jnp.concatenate([dst, jnp.full((pad,), -1, i32)]).reshape(ROWS, CH)
    val_rows = jnp.concatenate([jnp.ones((E,), jnp.float32),
                                jnp.zeros((pad,), jnp.float32)]).reshape(ROWS, CH)
    s_rows = jnp.concatenate([edge_label_index[0].astype(i32),
                              jnp.zeros((pad,), i32)]).reshape(SROWS, CH_S)
    t_rows = jnp.concatenate([edge_label_index[1].astype(i32),
                              jnp.zeros((pad,), i32)]).reshape(SROWS, CH_S)

    srcc, dstc, degp = _sc_compact(src_rows, dst_rows, val_rows)
    degsum = (degp[0] + degp[1])[:N].reshape(N, 1)
    dis, y0, y1 = _tc_prescale(degsum, x)

    srcc = srcc.reshape(NC, NS, NCH2, CH2)
    dstc = dstc.reshape(NC, NS, NCH2, CH2)
    acc0, acc1 = _sc_scatter(y0, y1, srcc, dstc)

    Wx = jnp.concatenate([W_xz[0], W_xh[0]], axis=1)
    Wt = jnp.concatenate([W_xz[1], W_xh[1]], axis=1)
    bc = jnp.concatenate([b_xz + b_hz, b_xh + b_hh]).reshape(1, 2 * D)
    wpost = W_post.sum(axis=1).reshape(1, D)
    h, u, hr = _tc_main(x, acc0, acc1, dis, Wx, Wt, bc, wpost)

    sc = _sc_score(u, hr, s_rows, t_rows)
    pred = sc.reshape(-1)[:E] + b_post.sum()
    return (pred, h)
